# Initial kernel scaffold; baseline (speedup 1.0000x reference)
import jax
import jax.numpy as jnp
from jax.experimental import pallas as pl


def kernel(values, feat_idx, row_ids, weight):
    raise NotImplementedError("write your pallas kernel here")


# speedup vs baseline: 8.7748x; 8.7748x over previous
import functools

import jax
import jax.numpy as jnp
from jax import lax
from jax.experimental import pallas as pl
from jax.experimental.pallas import tpu as pltpu
from jax.experimental.pallas import tpu_sc as plsc

NNZ = 1638400
VOCAB_SIZE = 1000000
KDIM = 16
NBATCH = 16384

NC = 2
NS = 16
NW = NC * NS
PER_W = NNZ // NW
CHUNK = 1024
NCHUNK = PER_W // CHUNK
GB = 128
NGB = CHUNK // GB
SB = 8
ROWS_PER_TILE = NBATCH // NS


def _sc_body(vals_hbm, feat2d_hbm, rids_hbm, weight_hbm, out_hbm,
             idx_v, vals_v, rids_v, rows_v, cbuf_v, crows_v, zbuf_v,
             acc_sh, sem):
    c_id = lax.axis_index("c")
    s_id = lax.axis_index("s")
    wid = c_id * NS + s_id
    base = wid * PER_W

    z16 = jnp.zeros((16,), jnp.float32)
    z16i = jnp.zeros((16,), jnp.int32)
    iota16 = lax.broadcasted_iota(jnp.int32, (16,), 0)
    lane0 = iota16 == 0

    def set_crow(n, r):
        plsc.store_scatter(crows_v, [z16i + n // SB, z16i + n % SB],
                           z16i + r, mask=lane0)

    def row_store(ref, r, lo, hi):
        plsc.store_scatter(ref, [z16i + r, iota16], lo)
        plsc.store_scatter(ref, [z16i + r, iota16 + 16], hi)

    def zero_zbuf(r, carry):
        row_store(zbuf_v, r, z16, z16)
        return carry

    lax.fori_loop(0, 128, zero_zbuf, 0)

    def zero_acc(t, carry):
        pltpu.sync_copy(zbuf_v, acc_sh.at[pl.ds(pl.multiple_of(s_id * ROWS_PER_TILE + t * 128, 128), 128)])
        return carry

    lax.fori_loop(0, ROWS_PER_TILE // 128, zero_acc, 0)
    plsc.subcore_barrier()

    def chunk_body(ci, carry):
        cb = pl.multiple_of(base + ci * CHUNK, CHUNK)
        pltpu.sync_copy(feat2d_hbm.at[pl.ds(pl.multiple_of(base // GB + ci * NGB, 8), NGB)], idx_v)
        pltpu.sync_copy(vals_hbm.at[pl.ds(cb, CHUNK)], vals_v)
        pltpu.sync_copy(rids_hbm.at[pl.ds(cb, CHUNK)], rids_v)
        descs = [
            pltpu.async_copy(weight_hbm.at[idx_v.at[j]],
                             rows_v.at[pl.ds(j * GB, GB)], sem)
            for j in range(NGB)
        ]
        for d in descs:
            d.wait()

        def group_body(g, st):
            cur_row, n, acc_seg, acc_sq = st
            gb = g * 16
            rid16 = rids_v[pl.ds(gb, 16)]
            v16 = vals_v[pl.ds(gb, 16)]
            for l in range(16):
                rid = rid16[l]
                v = v16[l]
                row = plsc.load_gather(rows_v, [z16i + (gb + l), iota16])
                w = row * v
                same = rid == cur_row

                @pl.when(jnp.logical_not(same))
                def _flush(n=n, cur_row=cur_row, acc_seg=acc_seg, acc_sq=acc_sq):
                    row_store(cbuf_v, n, acc_seg, acc_sq)
                    set_crow(n, cur_row)

                n = jnp.where(same, n, n + 1)
                acc_seg = jnp.where(same, acc_seg, z16) + w
                acc_sq = jnp.where(same, acc_sq, z16) + w * w
                cur_row = rid
            return (cur_row, n, acc_seg, acc_sq)

        init = (rids_v[pl.ds(0, 16)][0], jnp.int32(0), z16, z16)
        cur_row, n, acc_seg, acc_sq = lax.fori_loop(0, CHUNK // 16, group_body, init)
        row_store(cbuf_v, n, acc_seg, acc_sq)
        set_crow(n, cur_row)
        cnt = n + 1
        nb = (cnt + SB - 1) // SB

        def pad(k, c2):
            row_store(cbuf_v, k, z16, z16)
            set_crow(k, jnp.int32(0))
            return c2

        lax.fori_loop(cnt, nb * SB, pad, 0)

        def scat(j, c2):
            pltpu.sync_copy(cbuf_v.at[pl.ds(j * SB, SB)],
                            acc_sh.at[crows_v.at[j]], add=True)
            return c2

        lax.fori_loop(0, nb, scat, 0)
        return carry

    lax.fori_loop(0, NCHUNK, chunk_body, 0)
    plsc.subcore_barrier()
    out_base = pl.multiple_of(s_id * ROWS_PER_TILE, ROWS_PER_TILE)
    pltpu.sync_copy(acc_sh.at[pl.ds(out_base, ROWS_PER_TILE)],
                    out_hbm.at[c_id, pl.ds(out_base, ROWS_PER_TILE)])


_sc_kernel = functools.partial(
    pl.kernel,
    mesh=plsc.VectorSubcoreMesh(core_axis_name="c", subcore_axis_name="s",
                                num_cores=NC, num_subcores=NS),
    out_type=jax.ShapeDtypeStruct((NC, NBATCH, 32), jnp.float32),
    scratch_types=[
        pltpu.VMEM((NGB, GB), jnp.int32),
        pltpu.VMEM((CHUNK,), jnp.float32),
        pltpu.VMEM((CHUNK,), jnp.int32),
        pltpu.VMEM((CHUNK, KDIM), jnp.float32),
        pltpu.VMEM((CHUNK + SB, 32), jnp.float32),
        pltpu.VMEM(((CHUNK + SB) // SB, SB), jnp.int32),
        pltpu.VMEM((128, 32), jnp.float32),
        pltpu.VMEM_SHARED((NBATCH, 32), jnp.float32),
        pltpu.SemaphoreType.DMA,
    ],
    compiler_params=pltpu.CompilerParams(needs_layout_passes=False, use_tc_tiling_on_sc=False),
)(_sc_body)


def _combine_body(p_ref, o_ref):
    x = p_ref[...]
    p = x[0] + x[1]
    k = lax.broadcasted_iota(jnp.int32, (NBATCH, 32), 1)
    t = jnp.where(k < KDIM, p * p, -p)
    o_ref[...] = jnp.sum(t, axis=1, keepdims=True)


_combine = pl.pallas_call(
    _combine_body,
    out_shape=jax.ShapeDtypeStruct((NBATCH, 1), jnp.float32),
)


def kernel(values, feat_idx, row_ids, weight):
    feat2d = feat_idx.reshape(NNZ // GB, GB)
    part = _sc_kernel(values, feat2d, row_ids, weight)
    return _combine(part)
